# Initial kernel scaffold; baseline (speedup 1.0000x reference)
import jax
import jax.numpy as jnp
from jax.experimental import pallas as pl


def kernel(x, edge_index, W1, b1, W2, b2):
    raise NotImplementedError("write your pallas kernel here")


# speedup vs baseline: 23.2222x; 23.2222x over previous
import functools

import jax
import jax.numpy as jnp
from jax import lax
from jax.experimental import pallas as pl
from jax.experimental.pallas import tpu as pltpu
from jax.experimental.pallas import tpu_sc as plsc

N = 10000
E = 320000
D_IN = 128
D_HID = 128
D_OUT = 40
D_PAD = 128

NC = 2
NS = 16
NW = NC * NS
CHUNK = 80
NCHUNK = E // CHUNK
CPT = NCHUNK // NW
NSTAGE = 5
SPS = CPT // NSTAGE
ROWS_A = 632
ROWS_LAST = N - (NS - 1) * ROWS_A

_MESH = plsc.VectorSubcoreMesh(core_axis_name="c", subcore_axis_name="s")


def _zero_slice(zerosD, acc, s):
    @pl.when(s < NS - 1)
    def _():
        pltpu.sync_copy(zerosD, acc.at[pl.ds(s * ROWS_A, ROWS_A)])

    @pl.when(s == NS - 1)
    def _():
        pltpu.sync_copy(zerosD.at[pl.ds(0, ROWS_LAST)],
                        acc.at[pl.ds((NS - 1) * ROWS_A, ROWS_LAST)])


def _copy_out_slice(acc, out, c, s):
    @pl.when(s < NS - 1)
    def _():
        pltpu.sync_copy(acc.at[pl.ds(s * ROWS_A, ROWS_A)],
                        out.at[c, pl.ds(s * ROWS_A, ROWS_A)])

    @pl.when(s == NS - 1)
    def _():
        pltpu.sync_copy(acc.at[pl.ds((NS - 1) * ROWS_A, ROWS_LAST)],
                        out.at[c, pl.ds((NS - 1) * ROWS_A, ROWS_LAST)])


@functools.partial(
    pl.kernel,
    out_type=jax.ShapeDtypeStruct((NC, N, 16), jnp.float32),
    mesh=_MESH,
    scratch_types=[
        pltpu.VMEM_SHARED((N, 16), jnp.float32),
        pltpu.VMEM((CHUNK, 16), jnp.float32),
        pltpu.VMEM((CPT, CHUNK), jnp.int32),
    ],
)
def _deg_kernel(dst3d, ones16, zeros16, out, acc, onesb, idxb):
    c = lax.axis_index("c")
    s = lax.axis_index("s")
    wid = s * NC + c
    _zero_slice(zeros16, acc, s)
    pltpu.sync_copy(ones16, onesb)
    pltpu.sync_copy(dst3d.at[wid], idxb)
    plsc.subcore_barrier()

    def body(j, carry):
        pltpu.sync_copy(onesb, acc.at[idxb.at[j]], add=True)
        return carry

    lax.fori_loop(0, CPT, body, 0)
    plsc.subcore_barrier()
    _copy_out_slice(acc, out, c, s)


def _make_scatter(D):
    @functools.partial(
        pl.kernel,
        out_type=jax.ShapeDtypeStruct((NC, N, D), jnp.float32),
        mesh=_MESH,
        scratch_types=[
            pltpu.VMEM_SHARED((N, D), jnp.float32),
            pltpu.VMEM((SPS, CHUNK), jnp.int32),
            pltpu.VMEM((SPS, CHUNK), jnp.int32),
            pltpu.VMEM((2, CHUNK, D), jnp.float32),
            pltpu.SemaphoreType.DMA,
        ],
    )
    def _scatter_kernel(z, src4d, dst4d, zerosD, out, acc, srcb, dstb, rows, gsem):
        c = lax.axis_index("c")
        s = lax.axis_index("s")
        wid = s * NC + c
        _zero_slice(zerosD, acc, s)
        plsc.subcore_barrier()

        def stage_body(st, carry):
            pltpu.sync_copy(src4d.at[wid, st], srcb)
            pltpu.sync_copy(dst4d.at[wid, st], dstb)
            pltpu.async_copy(z.at[srcb.at[0]], rows.at[0], gsem)

            def body(j, carry2):
                pltpu.make_async_copy(z.at[srcb.at[j]], rows.at[j % 2],
                                      gsem).wait()

                @pl.when(j + 1 < SPS)
                def _():
                    pltpu.async_copy(z.at[srcb.at[j + 1]],
                                     rows.at[(j + 1) % 2], gsem)

                pltpu.sync_copy(rows.at[j % 2], acc.at[dstb.at[j]], add=True)
                return carry2

            lax.fori_loop(0, SPS, body, 0)
            return carry

        lax.fori_loop(0, NSTAGE, stage_body, 0)
        plsc.subcore_barrier()
        _copy_out_slice(acc, out, c, s)

    return _scatter_kernel


_scatter128 = _make_scatter(D_HID)


_R = 1000


def _dinv_of(degp_blk):
    deg = 1.0 + jnp.sum(degp_blk[0] + degp_blk[1], axis=1)
    return lax.rsqrt(deg)


def _z1_body(degp, x, w1, o):
    dinv = _dinv_of(degp[...])
    o[...] = dinv[:, None] * jnp.dot(x[...], w1[...],
                                     preferred_element_type=jnp.float32)


def _mid_body(degp, parts, z1, b1, w2, o):
    dinv = _dinv_of(degp[...])
    agg = parts[0] + parts[1] + z1[...]
    h = jnp.maximum(dinv[:, None] * agg + b1[...], 0.0)
    o[...] = dinv[:, None] * jnp.dot(h, w2[...],
                                     preferred_element_type=jnp.float32)


def _out_body(degp, parts, z2, b2, o):
    dinv = _dinv_of(degp[...])
    logits = dinv[:, None] * (parts[0] + parts[1] + z2[...]) + b2[...]
    m = jnp.max(logits, axis=1, keepdims=True)
    lse = jnp.log(jnp.sum(jnp.exp(logits - m), axis=1, keepdims=True)) + m
    o[...] = logits - lse


def _tc_call(body, out_d, in_specs):
    return pl.pallas_call(
        body,
        grid=(N // _R,),
        in_specs=in_specs,
        out_specs=pl.BlockSpec((_R, out_d), lambda i: (i, 0)),
        out_shape=jax.ShapeDtypeStruct((N, out_d), jnp.float32),
    )


_DEGP_SPEC = pl.BlockSpec((2, _R, 16), lambda i: (0, i, 0))


def _z1_call(degp, x, w1):
    return _tc_call(_z1_body, D_HID, [
        _DEGP_SPEC,
        pl.BlockSpec((_R, D_IN), lambda i: (i, 0)),
        pl.BlockSpec((D_IN, D_HID), lambda i: (0, 0)),
    ])(degp, x, w1)


def _mid_call(degp, parts, z1, b1, w2):
    return _tc_call(_mid_body, D_PAD, [
        _DEGP_SPEC,
        pl.BlockSpec((2, _R, D_HID), lambda i: (0, i, 0)),
        pl.BlockSpec((_R, D_HID), lambda i: (i, 0)),
        pl.BlockSpec((1, D_HID), lambda i: (0, 0)),
        pl.BlockSpec((D_HID, D_PAD), lambda i: (0, 0)),
    ])(degp, parts, z1, b1, w2)


def _out_call(degp, parts, z2, b2):
    return _tc_call(_out_body, D_PAD, [
        _DEGP_SPEC,
        pl.BlockSpec((2, _R, D_PAD), lambda i: (0, i, 0)),
        pl.BlockSpec((_R, D_PAD), lambda i: (i, 0)),
        pl.BlockSpec((1, D_PAD), lambda i: (0, 0)),
    ])(degp, parts, z2, b2)


def kernel(x, edge_index, W1, b1, W2, b2):
    src4d = edge_index[0].reshape(NW, NSTAGE, SPS, CHUNK)
    dst4d = edge_index[1].reshape(NW, NSTAGE, SPS, CHUNK)
    dst3d = edge_index[1].reshape(NW, CPT, CHUNK)

    zeros16 = jnp.zeros((ROWS_A, 16), jnp.float32)
    zeros128 = jnp.zeros((ROWS_A, D_HID), jnp.float32)
    ones16 = jnp.zeros((CHUNK, 16), jnp.float32).at[:, 0].set(1.0)

    w2pad = jnp.zeros((D_HID, D_PAD), jnp.float32).at[:, :D_OUT].set(W2)
    b1r = b1.reshape(1, D_HID)
    b2pad = jnp.full((1, D_PAD), -1e30, jnp.float32).at[0, :D_OUT].set(b2)

    degp = _deg_kernel(dst3d, ones16, zeros16)
    z1 = _z1_call(degp, x, W1)
    p1 = _scatter128(z1, src4d, dst4d, zeros128)
    z2 = _mid_call(degp, p1, z1, b1r, w2pad)
    q2 = _scatter128(z2, src4d, dst4d, zeros128)
    outp = _out_call(degp, q2, z2, b2pad)
    return outp[:, :D_OUT]


# speedup vs baseline: 30.9349x; 1.3321x over previous
import functools

import jax
import jax.numpy as jnp
from jax import lax
from jax.experimental import pallas as pl
from jax.experimental.pallas import tpu as pltpu
from jax.experimental.pallas import tpu_sc as plsc

N = 10000
E = 320000
D_IN = 128
D_HID = 128
D_OUT = 40
D_PAD = 128

NC = 2
NS = 16
NW = NC * NS
CHUNK = 80
NCHUNK = E // CHUNK
CPT = NCHUNK // NW
NSTAGE = 5
SPS = CPT // NSTAGE
ROWS_A = 632
ROWS_LAST = N - (NS - 1) * ROWS_A

_MESH = plsc.VectorSubcoreMesh(core_axis_name="c", subcore_axis_name="s")


def _zero_slice(zerosD, acc, s):
    @pl.when(s < NS - 1)
    def _():
        pltpu.sync_copy(zerosD, acc.at[pl.ds(s * ROWS_A, ROWS_A)])

    @pl.when(s == NS - 1)
    def _():
        pltpu.sync_copy(zerosD.at[pl.ds(0, ROWS_LAST)],
                        acc.at[pl.ds((NS - 1) * ROWS_A, ROWS_LAST)])


def _copy_out_slice(acc, out, c, s):
    @pl.when(s < NS - 1)
    def _():
        pltpu.sync_copy(acc.at[pl.ds(s * ROWS_A, ROWS_A)],
                        out.at[c, pl.ds(s * ROWS_A, ROWS_A)])

    @pl.when(s == NS - 1)
    def _():
        pltpu.sync_copy(acc.at[pl.ds((NS - 1) * ROWS_A, ROWS_LAST)],
                        out.at[c, pl.ds((NS - 1) * ROWS_A, ROWS_LAST)])


@functools.partial(
    pl.kernel,
    out_type=jax.ShapeDtypeStruct((NC, N, 16), jnp.float32),
    mesh=_MESH,
    scratch_types=[
        pltpu.VMEM_SHARED((N, 16), jnp.float32),
        pltpu.VMEM((CHUNK, 16), jnp.float32),
        pltpu.VMEM((CPT, CHUNK), jnp.int32),
        pltpu.SemaphoreType.DMA,
    ],
)
def _deg_kernel(dst3d, ones16, zeros16, out, acc, onesb, idxb, dsem):
    c = lax.axis_index("c")
    s = lax.axis_index("s")
    wid = s * NC + c
    _zero_slice(zeros16, acc, s)
    pltpu.sync_copy(ones16, onesb)
    pltpu.sync_copy(dst3d.at[wid], idxb)
    plsc.subcore_barrier()

    def body(j, carry):
        pltpu.async_copy(onesb, acc.at[idxb.at[j]], dsem, add=True)

        @pl.when(j >= 8)
        def _():
            pltpu.make_async_copy(onesb, acc.at[idxb.at[j - 8]], dsem).wait()

        return carry

    lax.fori_loop(0, CPT, body, 0)

    def drain(j, carry):
        pltpu.make_async_copy(onesb, acc.at[idxb.at[j]], dsem).wait()
        return carry

    lax.fori_loop(CPT - 8, CPT, drain, 0)
    plsc.subcore_barrier()
    _copy_out_slice(acc, out, c, s)


def _make_scatter(D):
    @functools.partial(
        pl.kernel,
        out_type=jax.ShapeDtypeStruct((NC, N, D), jnp.float32),
        mesh=_MESH,
        scratch_types=[
            pltpu.VMEM_SHARED((N, D), jnp.float32),
            pltpu.VMEM((SPS, CHUNK), jnp.int32),
            pltpu.VMEM((SPS, CHUNK), jnp.int32),
            pltpu.VMEM((3, CHUNK, D), jnp.float32),
            pltpu.SemaphoreType.DMA((3,)),
            pltpu.SemaphoreType.DMA((3,)),
        ],
    )
    def _scatter_kernel(z, src4d, dst4d, zerosD, out, acc, srcb, dstb, rows,
                        gsem, ssem):
        c = lax.axis_index("c")
        s = lax.axis_index("s")
        wid = s * NC + c
        _zero_slice(zerosD, acc, s)
        plsc.subcore_barrier()

        def stage_body(st, carry):
            pltpu.sync_copy(src4d.at[wid, st], srcb)
            pltpu.sync_copy(dst4d.at[wid, st], dstb)
            pltpu.async_copy(z.at[srcb.at[0]], rows.at[0], gsem.at[0])
            pltpu.async_copy(z.at[srcb.at[1]], rows.at[1], gsem.at[1])

            def body(j, carry2):
                b = j % 3
                pltpu.make_async_copy(z.at[srcb.at[j]], rows.at[b],
                                      gsem.at[b]).wait()
                pltpu.async_copy(rows.at[b], acc.at[dstb.at[j]], ssem.at[b],
                                 add=True)

                @pl.when(j >= 1)
                def _():
                    pltpu.make_async_copy(rows.at[(j - 1) % 3],
                                          acc.at[dstb.at[j - 1]],
                                          ssem.at[(j - 1) % 3]).wait()

                @pl.when(j + 2 < SPS)
                def _():
                    pltpu.async_copy(z.at[srcb.at[j + 2]],
                                     rows.at[(j + 2) % 3],
                                     gsem.at[(j + 2) % 3])

                return carry2

            lax.fori_loop(0, SPS, body, 0)
            pltpu.make_async_copy(rows.at[(SPS - 1) % 3],
                                  acc.at[dstb.at[SPS - 1]],
                                  ssem.at[(SPS - 1) % 3]).wait()
            return carry

        lax.fori_loop(0, NSTAGE, stage_body, 0)
        plsc.subcore_barrier()
        _copy_out_slice(acc, out, c, s)

    return _scatter_kernel


_scatter128 = _make_scatter(D_HID)


_R = 1000


def _dinv_of(degp_blk):
    deg = 1.0 + jnp.sum(degp_blk[0] + degp_blk[1], axis=1)
    return lax.rsqrt(deg)


def _z1_body(degp, x, w1, o):
    dinv = _dinv_of(degp[...])
    o[...] = dinv[:, None] * jnp.dot(x[...], w1[...],
                                     preferred_element_type=jnp.float32)


def _mid_body(degp, parts, z1, b1, w2, o):
    dinv = _dinv_of(degp[...])
    agg = parts[0] + parts[1] + z1[...]
    h = jnp.maximum(dinv[:, None] * agg + b1[...], 0.0)
    o[...] = dinv[:, None] * jnp.dot(h, w2[...],
                                     preferred_element_type=jnp.float32)


def _out_body(degp, parts, z2, b2, o):
    dinv = _dinv_of(degp[...])
    logits = dinv[:, None] * (parts[0] + parts[1] + z2[...]) + b2[...]
    m = jnp.max(logits, axis=1, keepdims=True)
    lse = jnp.log(jnp.sum(jnp.exp(logits - m), axis=1, keepdims=True)) + m
    o[...] = logits - lse


def _tc_call(body, out_d, in_specs):
    return pl.pallas_call(
        body,
        grid=(N // _R,),
        in_specs=in_specs,
        out_specs=pl.BlockSpec((_R, out_d), lambda i: (i, 0)),
        out_shape=jax.ShapeDtypeStruct((N, out_d), jnp.float32),
    )


_DEGP_SPEC = pl.BlockSpec((2, _R, 16), lambda i: (0, i, 0))


def _z1_call(degp, x, w1):
    return _tc_call(_z1_body, D_HID, [
        _DEGP_SPEC,
        pl.BlockSpec((_R, D_IN), lambda i: (i, 0)),
        pl.BlockSpec((D_IN, D_HID), lambda i: (0, 0)),
    ])(degp, x, w1)


def _mid_call(degp, parts, z1, b1, w2):
    return _tc_call(_mid_body, D_PAD, [
        _DEGP_SPEC,
        pl.BlockSpec((2, _R, D_HID), lambda i: (0, i, 0)),
        pl.BlockSpec((_R, D_HID), lambda i: (i, 0)),
        pl.BlockSpec((1, D_HID), lambda i: (0, 0)),
        pl.BlockSpec((D_HID, D_PAD), lambda i: (0, 0)),
    ])(degp, parts, z1, b1, w2)


def _out_call(degp, parts, z2, b2):
    return _tc_call(_out_body, D_PAD, [
        _DEGP_SPEC,
        pl.BlockSpec((2, _R, D_PAD), lambda i: (0, i, 0)),
        pl.BlockSpec((_R, D_PAD), lambda i: (i, 0)),
        pl.BlockSpec((1, D_PAD), lambda i: (0, 0)),
    ])(degp, parts, z2, b2)


def kernel(x, edge_index, W1, b1, W2, b2):
    src4d = edge_index[0].reshape(NW, NSTAGE, SPS, CHUNK)
    dst4d = edge_index[1].reshape(NW, NSTAGE, SPS, CHUNK)
    dst3d = edge_index[1].reshape(NW, CPT, CHUNK)

    zeros16 = jnp.zeros((ROWS_A, 16), jnp.float32)
    zeros128 = jnp.zeros((ROWS_A, D_HID), jnp.float32)
    ones16 = jnp.zeros((CHUNK, 16), jnp.float32).at[:, 0].set(1.0)

    w2pad = jnp.zeros((D_HID, D_PAD), jnp.float32).at[:, :D_OUT].set(W2)
    b1r = b1.reshape(1, D_HID)
    b2pad = jnp.full((1, D_PAD), -1e30, jnp.float32).at[0, :D_OUT].set(b2)

    degp = _deg_kernel(dst3d, ones16, zeros16)
    z1 = _z1_call(degp, x, W1)
    p1 = _scatter128(z1, src4d, dst4d, zeros128)
    z2 = _mid_call(degp, p1, z1, b1r, w2pad)
    q2 = _scatter128(z2, src4d, dst4d, zeros128)
    outp = _out_call(degp, q2, z2, b2pad)
    return outp[:, :D_OUT]


# speedup vs baseline: 35.4628x; 1.1464x over previous
import functools

import jax
import jax.numpy as jnp
from jax import lax
from jax.experimental import pallas as pl
from jax.experimental.pallas import tpu as pltpu
from jax.experimental.pallas import tpu_sc as plsc

N = 10000
E = 320000
D_IN = 128
D_HID = 128
D_OUT = 40
D_PAD = 48

NC = 2
NS = 16
NW = NC * NS
CHUNK = 80
NCHUNK = E // CHUNK
CPT = NCHUNK // NW
NSTAGE = 5
SPS = CPT // NSTAGE
ROWS_A = 632
ROWS_LAST = N - (NS - 1) * ROWS_A

_MESH = plsc.VectorSubcoreMesh(core_axis_name="c", subcore_axis_name="s")


def _zero_slice(zerosD, acc, s):
    @pl.when(s < NS - 1)
    def _():
        pltpu.sync_copy(zerosD, acc.at[pl.ds(s * ROWS_A, ROWS_A)])

    @pl.when(s == NS - 1)
    def _():
        pltpu.sync_copy(zerosD.at[pl.ds(0, ROWS_LAST)],
                        acc.at[pl.ds((NS - 1) * ROWS_A, ROWS_LAST)])


def _copy_out_slice(acc, out, c, s):
    @pl.when(s < NS - 1)
    def _():
        pltpu.sync_copy(acc.at[pl.ds(s * ROWS_A, ROWS_A)],
                        out.at[c, pl.ds(s * ROWS_A, ROWS_A)])

    @pl.when(s == NS - 1)
    def _():
        pltpu.sync_copy(acc.at[pl.ds((NS - 1) * ROWS_A, ROWS_LAST)],
                        out.at[c, pl.ds((NS - 1) * ROWS_A, ROWS_LAST)])


@functools.partial(
    pl.kernel,
    out_type=jax.ShapeDtypeStruct((NC, N, 16), jnp.float32),
    mesh=_MESH,
    scratch_types=[
        pltpu.VMEM_SHARED((N, 16), jnp.float32),
        pltpu.VMEM((CHUNK, 16), jnp.float32),
        pltpu.VMEM((CPT, CHUNK), jnp.int32),
        pltpu.SemaphoreType.DMA,
    ],
)
def _deg_kernel(dst3d, ones16, zeros16, out, acc, onesb, idxb, dsem):
    c = lax.axis_index("c")
    s = lax.axis_index("s")
    wid = s * NC + c
    _zero_slice(zeros16, acc, s)
    pltpu.sync_copy(ones16, onesb)
    pltpu.sync_copy(dst3d.at[wid], idxb)
    plsc.subcore_barrier()

    def body(j, carry):
        pltpu.async_copy(onesb, acc.at[idxb.at[j]], dsem, add=True)

        @pl.when(j >= 8)
        def _():
            pltpu.make_async_copy(onesb, acc.at[idxb.at[j - 8]], dsem).wait()

        return carry

    lax.fori_loop(0, CPT, body, 0)

    def drain(j, carry):
        pltpu.make_async_copy(onesb, acc.at[idxb.at[j]], dsem).wait()
        return carry

    lax.fori_loop(CPT - 8, CPT, drain, 0)
    plsc.subcore_barrier()
    _copy_out_slice(acc, out, c, s)


def _make_scatter(D):
    @functools.partial(
        pl.kernel,
        out_type=jax.ShapeDtypeStruct((NC, N, D), jnp.float32),
        mesh=_MESH,
        scratch_types=[
            pltpu.VMEM_SHARED((N, D), jnp.float32),
            pltpu.VMEM((SPS, CHUNK), jnp.int32),
            pltpu.VMEM((SPS, CHUNK), jnp.int32),
            pltpu.VMEM((3, CHUNK, D), jnp.float32),
            pltpu.SemaphoreType.DMA((3,)),
            pltpu.SemaphoreType.DMA((3,)),
        ],
    )
    def _scatter_kernel(z, src4d, dst4d, zerosD, out, acc, srcb, dstb, rows,
                        gsem, ssem):
        c = lax.axis_index("c")
        s = lax.axis_index("s")
        wid = s * NC + c
        _zero_slice(zerosD, acc, s)
        plsc.subcore_barrier()

        def stage_body(st, carry):
            pltpu.sync_copy(src4d.at[wid, st], srcb)
            pltpu.sync_copy(dst4d.at[wid, st], dstb)
            pltpu.async_copy(z.at[srcb.at[0]], rows.at[0], gsem.at[0])
            pltpu.async_copy(z.at[srcb.at[1]], rows.at[1], gsem.at[1])

            def body(j, carry2):
                b = j % 3
                pltpu.make_async_copy(z.at[srcb.at[j]], rows.at[b],
                                      gsem.at[b]).wait()
                pltpu.async_copy(rows.at[b], acc.at[dstb.at[j]], ssem.at[b],
                                 add=True)

                @pl.when(j >= 1)
                def _():
                    pltpu.make_async_copy(rows.at[(j - 1) % 3],
                                          acc.at[dstb.at[j - 1]],
                                          ssem.at[(j - 1) % 3]).wait()

                @pl.when(j + 2 < SPS)
                def _():
                    pltpu.async_copy(z.at[srcb.at[j + 2]],
                                     rows.at[(j + 2) % 3],
                                     gsem.at[(j + 2) % 3])

                return carry2

            lax.fori_loop(0, SPS, body, 0)
            pltpu.make_async_copy(rows.at[(SPS - 1) % 3],
                                  acc.at[dstb.at[SPS - 1]],
                                  ssem.at[(SPS - 1) % 3]).wait()
            return carry

        lax.fori_loop(0, NSTAGE, stage_body, 0)
        plsc.subcore_barrier()
        _copy_out_slice(acc, out, c, s)

    return _scatter_kernel


_scatter128 = _make_scatter(D_HID)


@functools.partial(
    pl.kernel,
    out_type=jax.ShapeDtypeStruct((NC, N, D_PAD), jnp.float32),
    mesh=_MESH,
    scratch_types=[
        pltpu.VMEM_SHARED((N, D_PAD), jnp.float32),
        pltpu.VMEM_SHARED((N, D_PAD), jnp.float32),
        pltpu.VMEM((SPS, CHUNK), jnp.int32),
        pltpu.VMEM((SPS, CHUNK), jnp.int32),
        pltpu.VMEM((3, CHUNK, D_PAD), jnp.float32),
        pltpu.SemaphoreType.DMA((3,)),
        pltpu.SemaphoreType.DMA((3,)),
    ],
)
def _scatter_spmem(z, src4d, dst4d, zerosD, out, zs, acc, srcb, dstb, rows,
                   gsem, ssem):
    c = lax.axis_index("c")
    s = lax.axis_index("s")
    wid = s * NC + c
    _zero_slice(zerosD, acc, s)

    @pl.when(s < NS - 1)
    def _():
        pltpu.sync_copy(z.at[pl.ds(s * ROWS_A, ROWS_A)],
                        zs.at[pl.ds(s * ROWS_A, ROWS_A)])

    @pl.when(s == NS - 1)
    def _():
        pltpu.sync_copy(z.at[pl.ds((NS - 1) * ROWS_A, ROWS_LAST)],
                        zs.at[pl.ds((NS - 1) * ROWS_A, ROWS_LAST)])

    plsc.subcore_barrier()

    def stage_body(st, carry):
        pltpu.sync_copy(src4d.at[wid, st], srcb)
        pltpu.sync_copy(dst4d.at[wid, st], dstb)
        pltpu.async_copy(zs.at[srcb.at[0]], rows.at[0], gsem.at[0])
        pltpu.async_copy(zs.at[srcb.at[1]], rows.at[1], gsem.at[1])

        def body(j, carry2):
            b = j % 3
            pltpu.make_async_copy(zs.at[srcb.at[j]], rows.at[b],
                                  gsem.at[b]).wait()
            pltpu.async_copy(rows.at[b], acc.at[dstb.at[j]], ssem.at[b],
                             add=True)

            @pl.when(j >= 1)
            def _():
                pltpu.make_async_copy(rows.at[(j - 1) % 3],
                                      acc.at[dstb.at[j - 1]],
                                      ssem.at[(j - 1) % 3]).wait()

            @pl.when(j + 2 < SPS)
            def _():
                pltpu.async_copy(zs.at[srcb.at[j + 2]], rows.at[(j + 2) % 3],
                                 gsem.at[(j + 2) % 3])

            return carry2

        lax.fori_loop(0, SPS, body, 0)
        pltpu.make_async_copy(rows.at[(SPS - 1) % 3],
                              acc.at[dstb.at[SPS - 1]],
                              ssem.at[(SPS - 1) % 3]).wait()
        return carry

    lax.fori_loop(0, NSTAGE, stage_body, 0)
    plsc.subcore_barrier()
    _copy_out_slice(acc, out, c, s)


_R = 1000


def _dinv_of(degp_blk):
    deg = 1.0 + jnp.sum(degp_blk[0] + degp_blk[1], axis=1)
    return lax.rsqrt(deg)


def _z1_body(degp, x, w1, o):
    dinv = _dinv_of(degp[...])
    o[...] = dinv[:, None] * jnp.dot(x[...], w1[...],
                                     preferred_element_type=jnp.float32)


def _mid_body(degp, parts, z1, b1, w2, o):
    dinv = _dinv_of(degp[...])
    agg = parts[0] + parts[1] + z1[...]
    h = jnp.maximum(dinv[:, None] * agg + b1[...], 0.0)
    o[...] = dinv[:, None] * jnp.dot(h, w2[...],
                                     preferred_element_type=jnp.float32)


def _out_body(degp, parts, z2, b2, o):
    dinv = _dinv_of(degp[...])
    logits = dinv[:, None] * (parts[0] + parts[1] + z2[...]) + b2[...]
    m = jnp.max(logits, axis=1, keepdims=True)
    lse = jnp.log(jnp.sum(jnp.exp(logits - m), axis=1, keepdims=True)) + m
    o[...] = logits - lse


def _tc_call(body, out_d, in_specs):
    return pl.pallas_call(
        body,
        grid=(N // _R,),
        in_specs=in_specs,
        out_specs=pl.BlockSpec((_R, out_d), lambda i: (i, 0)),
        out_shape=jax.ShapeDtypeStruct((N, out_d), jnp.float32),
    )


_DEGP_SPEC = pl.BlockSpec((2, _R, 16), lambda i: (0, i, 0))


def _z1_call(degp, x, w1):
    return _tc_call(_z1_body, D_HID, [
        _DEGP_SPEC,
        pl.BlockSpec((_R, D_IN), lambda i: (i, 0)),
        pl.BlockSpec((D_IN, D_HID), lambda i: (0, 0)),
    ])(degp, x, w1)


def _mid_call(degp, parts, z1, b1, w2):
    return _tc_call(_mid_body, D_PAD, [
        _DEGP_SPEC,
        pl.BlockSpec((2, _R, D_HID), lambda i: (0, i, 0)),
        pl.BlockSpec((_R, D_HID), lambda i: (i, 0)),
        pl.BlockSpec((1, D_HID), lambda i: (0, 0)),
        pl.BlockSpec((D_HID, D_PAD), lambda i: (0, 0)),
    ])(degp, parts, z1, b1, w2)


def _out_call(degp, parts, z2, b2):
    return _tc_call(_out_body, D_PAD, [
        _DEGP_SPEC,
        pl.BlockSpec((2, _R, D_PAD), lambda i: (0, i, 0)),
        pl.BlockSpec((_R, D_PAD), lambda i: (i, 0)),
        pl.BlockSpec((1, D_PAD), lambda i: (0, 0)),
    ])(degp, parts, z2, b2)


def kernel(x, edge_index, W1, b1, W2, b2):
    src4d = edge_index[0].reshape(NW, NSTAGE, SPS, CHUNK)
    dst4d = edge_index[1].reshape(NW, NSTAGE, SPS, CHUNK)
    dst3d = edge_index[1].reshape(NW, CPT, CHUNK)

    zeros16 = jnp.zeros((ROWS_A, 16), jnp.float32)
    zeros128 = jnp.zeros((ROWS_A, D_HID), jnp.float32)
    zeros48 = jnp.zeros((ROWS_A, D_PAD), jnp.float32)
    ones16 = jnp.zeros((CHUNK, 16), jnp.float32).at[:, 0].set(1.0)

    w2pad = jnp.zeros((D_HID, D_PAD), jnp.float32).at[:, :D_OUT].set(W2)
    b1r = b1.reshape(1, D_HID)
    b2pad = jnp.full((1, D_PAD), -1e30, jnp.float32).at[0, :D_OUT].set(b2)

    degp = _deg_kernel(dst3d, ones16, zeros16)
    z1 = _z1_call(degp, x, W1)
    p1 = _scatter128(z1, src4d, dst4d, zeros128)
    z2 = _mid_call(degp, p1, z1, b1r, w2pad)
    q2 = _scatter_spmem(z2, src4d, dst4d, zeros48)
    outp = _out_call(degp, q2, z2, b2pad)
    return outp[:, :D_OUT]
